# baseline (speedup 1.0000x reference)
import functools

import jax
import jax.numpy as jnp
from jax import lax
from jax.experimental import pallas as pl
from jax.experimental.pallas import tpu as pltpu
from jax.experimental.pallas import tpu_sc as plsc

N_LAYERS = 6
VOCAB = 100000
HIDDEN = 64
B = 1024
L = 50
TOK = B * L
NW = 32
PER_W = TOK // NW
CHUNK = 800
NCHUNK = PER_W // CHUNK
NBUF = 2
NSTEP = N_LAYERS * NCHUNK


def _emb_body(ids_hbm, tab_hbm, out_hbm, idx_v, rows_v, gsems, wsems):
    wid = lax.axis_index("s") * 2 + lax.axis_index("c")
    base = wid * PER_W
    pltpu.sync_copy(ids_hbm.at[wid], idx_v)

    gathers = [None] * NBUF
    writes = [None] * NBUF

    def step_of(s):
        return s // NCHUNK, s % NCHUNK

    for s in range(NSTEP + 1):
        if s < NSTEP:
            b = s % NBUF
            if writes[b] is not None:
                writes[b][0].wait()
                writes[b][1].wait()
            layer, c = step_of(s)
            gathers[b] = pltpu.async_copy(
                tab_hbm.at[layer].at[idx_v.at[c]], rows_v.at[b], gsems[b])
        if s >= 1:
            pb = (s - 1) % NBUF
            layer, c = step_of(s - 1)
            gathers[pb].wait()
            off = base + c * CHUNK
            w0 = pltpu.async_copy(
                rows_v.at[pb], out_hbm.at[layer, pl.ds(off, CHUNK)],
                wsems[pb])
            w1 = pltpu.async_copy(
                rows_v.at[pb],
                out_hbm.at[2 * N_LAYERS - 1 - layer, pl.ds(off, CHUNK)],
                wsems[pb])
            writes[pb] = (w0, w1)

    for b in range(NBUF):
        if writes[b] is not None:
            writes[b][0].wait()
            writes[b][1].wait()


@functools.partial(
    pl.kernel,
    mesh=plsc.VectorSubcoreMesh(core_axis_name="c", subcore_axis_name="s"),
    compiler_params=pltpu.CompilerParams(use_tc_tiling_on_sc=False),
    out_type=jax.ShapeDtypeStruct((2 * N_LAYERS, TOK, HIDDEN), jnp.float32),
    scratch_types=[
        pltpu.VMEM((NCHUNK, CHUNK), jnp.int32),
        pltpu.VMEM((NBUF, CHUNK, HIDDEN), jnp.float32),
        [pltpu.SemaphoreType.DMA] * NBUF,
        [pltpu.SemaphoreType.DMA] * NBUF,
    ],
)
def _emb_kernel(ids_hbm, tab_hbm, out_hbm, idx_v, rows_v, gsems, wsems):
    _emb_body(ids_hbm, tab_hbm, out_hbm, idx_v, rows_v, gsems, wsems)


def kernel(input_ids, tables):
    ids = input_ids.reshape(NW, NCHUNK, CHUNK)
    out = _emb_kernel(ids, tables)
    return out.reshape(2 * N_LAYERS, B, L, HIDDEN)
